# baseline (speedup 1.0000x reference)
import jax
import jax.numpy as jnp
from jax.experimental import pallas as pl
from jax.experimental.pallas import tpu as pltpu

_N, _P, _S = 16, 4096, 64
_ROWS = _N * _P // 2
_BLK = 2048


def _tent_body(d_ref, s_ref, wm_ref, wh_ref, o_ref):
    d = d_ref[...]
    sam = s_ref[...]
    m = jnp.dot(d, wm_ref[...], preferred_element_type=jnp.float32)
    h = jnp.dot(d, wh_ref[...], preferred_element_type=jnp.float32)
    o_ref[...] = jnp.maximum(h - jnp.abs(sam - m), 0.0)


def kernel(diagrams, samples):
    d4 = diagrams.reshape(_ROWS, 4)
    d8 = jnp.pad(d4, ((0, 0), (0, 4)))
    sam2 = jnp.concatenate([samples, samples])[None, :]
    half = jnp.ones((_S,), jnp.float32) * 0.5
    zero = jnp.zeros((_S,), jnp.float32)
    wm = jnp.stack([
        jnp.concatenate([half, zero]),
        jnp.concatenate([half, zero]),
        jnp.concatenate([zero, half]),
        jnp.concatenate([zero, half]),
        zero2 := jnp.zeros((2 * _S,), jnp.float32),
        zero2, zero2, zero2,
    ])
    wh = jnp.stack([
        jnp.concatenate([-half, zero]),
        jnp.concatenate([half, zero]),
        jnp.concatenate([zero, -half]),
        jnp.concatenate([zero, half]),
        zero2, zero2, zero2, zero2,
    ])
    out = pl.pallas_call(
        _tent_body,
        grid=(_ROWS // _BLK,),
        in_specs=[
            pl.BlockSpec((_BLK, 8), lambda i: (i, 0)),
            pl.BlockSpec((1, 2 * _S), lambda i: (0, 0)),
            pl.BlockSpec((8, 2 * _S), lambda i: (0, 0)),
            pl.BlockSpec((8, 2 * _S), lambda i: (0, 0)),
        ],
        out_specs=pl.BlockSpec((_BLK, 2 * _S), lambda i: (i, 0)),
        out_shape=jax.ShapeDtypeStruct((_ROWS, 2 * _S), jnp.float32),
    )(d8, sam2, wm, wh)
    return out.reshape(_N, _P, _S)


# speedup vs baseline: 1.6418x; 1.6418x over previous
import jax
import jax.numpy as jnp
from jax.experimental import pallas as pl
from jax.experimental.pallas import tpu as pltpu

_N, _P, _S = 16, 4096, 64
_BP = 4096


def _tent_body(d_ref, s_ref, o_ref):
    d = d_ref[0]
    sam = s_ref[...].reshape(1, _S)
    x = d[:, 0:1]
    y = d[:, 1:2]
    m = jnp.broadcast_to(0.5 * (x + y), (_BP, _S))
    h = jnp.broadcast_to(0.5 * (y - x), (_BP, _S))
    o_ref[0] = jnp.maximum(h - jnp.abs(sam - m), 0.0)


def kernel(diagrams, samples):
    return pl.pallas_call(
        _tent_body,
        grid=(_N * _P // _BP,),
        in_specs=[
            pl.BlockSpec((1, _BP, 2), lambda i: (i, 0, 0)),
            pl.BlockSpec((_S,), lambda i: (0,)),
        ],
        out_specs=pl.BlockSpec((1, _BP, _S), lambda i: (i, 0, 0)),
        out_shape=jax.ShapeDtypeStruct((_N, _P, _S), jnp.float32),
    )(diagrams.reshape(_N * _P // _BP, _BP, 2), samples)


# speedup vs baseline: 2.0584x; 1.2537x over previous
import jax
import jax.numpy as jnp
from jax.experimental import pallas as pl
from jax.experimental.pallas import tpu as pltpu

_N, _P, _S = 16, 4096, 64
_BP = 4096


def _tent_body(d_ref, s_ref, o_ref):
    d = d_ref[0]
    sam = s_ref[...].reshape(1, _S)
    x = d[:, 0:1]
    y = d[:, 1:2]
    m = jnp.broadcast_to(0.5 * (x + y), (_BP, _S))
    h = jnp.broadcast_to(0.5 * (y - x), (_BP, _S))
    o_ref[0] = jnp.zeros((_BP, _S), jnp.float32) + sam


def kernel(diagrams, samples):
    return pl.pallas_call(
        _tent_body,
        grid=(_N * _P // _BP,),
        in_specs=[
            pl.BlockSpec((1, _BP, 2), lambda i: (i, 0, 0)),
            pl.BlockSpec((_S,), lambda i: (0,)),
        ],
        out_specs=pl.BlockSpec((1, _BP, _S), lambda i: (i, 0, 0)),
        out_shape=jax.ShapeDtypeStruct((_N, _P, _S), jnp.float32),
    )(diagrams.reshape(_N * _P // _BP, _BP, 2), samples)


# speedup vs baseline: 10.1803x; 4.9458x over previous
import jax
import jax.numpy as jnp
from jax.experimental import pallas as pl
from jax.experimental.pallas import tpu as pltpu

_N, _P, _S = 16, 4096, 64


def _tent_body(d_ref, s_ref, o_ref):
    d = d_ref[0]
    x = d[0:1, :]
    y = d[1:2, :]
    m = 0.5 * (x + y)
    h = 0.5 * (y - x)
    sam = s_ref[...].reshape(_S, 1)
    o_ref[0] = jnp.maximum(h - jnp.abs(sam - m), 0.0)


def kernel(diagrams, samples):
    dt = jnp.transpose(diagrams, (0, 2, 1))
    out_t = pl.pallas_call(
        _tent_body,
        grid=(_N,),
        in_specs=[
            pl.BlockSpec((1, 2, _P), lambda i: (i, 0, 0)),
            pl.BlockSpec((_S,), lambda i: (0,)),
        ],
        out_specs=pl.BlockSpec((1, _S, _P), lambda i: (i, 0, 0)),
        out_shape=jax.ShapeDtypeStruct((_N, _S, _P), jnp.float32),
    )(dt, samples)
    return jnp.transpose(out_t, (0, 2, 1))


# speedup vs baseline: 14.3530x; 1.4099x over previous
import jax
import jax.numpy as jnp
from jax.experimental import pallas as pl
from jax.experimental.pallas import tpu as pltpu

_N, _P, _S = 16, 4096, 64


def _tent_body(d_ref, s_ref, o_ref):
    sam = s_ref[...].reshape(_S, 1)
    for k in range(2):
        d = d_ref[k]
        x = d[0:1, :]
        y = d[1:2, :]
        m = 0.5 * (x + y)
        h = 0.5 * (y - x)
        o_ref[k] = jnp.maximum(h - jnp.abs(sam - m), 0.0)


def kernel(diagrams, samples):
    dt = jnp.transpose(diagrams, (0, 2, 1))
    out_t = pl.pallas_call(
        _tent_body,
        grid=(_N // 2,),
        in_specs=[
            pl.BlockSpec((2, 2, _P), lambda i: (i, 0, 0)),
            pl.BlockSpec((_S,), lambda i: (0,)),
        ],
        out_specs=pl.BlockSpec((2, _S, _P), lambda i: (i, 0, 0)),
        out_shape=jax.ShapeDtypeStruct((_N, _S, _P), jnp.float32),
    )(dt, samples)
    return jnp.transpose(out_t, (0, 2, 1))


# speedup vs baseline: 16.5006x; 1.1496x over previous
import jax
import jax.numpy as jnp
from jax.experimental import pallas as pl
from jax.experimental.pallas import tpu as pltpu

_N, _P, _S = 16, 4096, 64


def _tent_body(d_ref, s_ref, o_ref):
    sam = s_ref[...].reshape(_S, 1)
    for k in range(4):
        d = d_ref[k]
        x = d[0:1, :]
        y = d[1:2, :]
        m = 0.5 * (x + y)
        h = 0.5 * (y - x)
        o_ref[k] = jnp.maximum(h - jnp.abs(sam - m), 0.0)


def kernel(diagrams, samples):
    dt = jnp.transpose(diagrams, (0, 2, 1))
    out_t = pl.pallas_call(
        _tent_body,
        grid=(_N // 4,),
        in_specs=[
            pl.BlockSpec((4, 2, _P), lambda i: (i, 0, 0)),
            pl.BlockSpec((_S,), lambda i: (0,)),
        ],
        out_specs=pl.BlockSpec((4, _S, _P), lambda i: (i, 0, 0)),
        out_shape=jax.ShapeDtypeStruct((_N, _S, _P), jnp.float32),
    )(dt, samples)
    return jnp.transpose(out_t, (0, 2, 1))
